# baseline (speedup 1.0000x reference)
import jax
import jax.numpy as jnp
from jax.experimental import pallas as pl
from jax.experimental.pallas import tpu as pltpu

DIM = 32
N_EMBED = 8192
N_ROWS = 8192
COMMITMENT_COST = 0.25
BLOCK_ROWS = 1024
N_BLOCKS = N_ROWS // BLOCK_ROWS


def _vq_tail_kernel(idx_ref, x_ref, w_ref,
                    qst_ref, loss_ref, perp_ref,
                    counts_ref, sumsq_ref):
    i = pl.program_id(0)

    x = x_ref[...]
    w = w_ref[...]
    idx = idx_ref[...]

    iota = jax.lax.broadcasted_iota(jnp.int32, (BLOCK_ROWS, N_EMBED), 1)
    onehot = (iota == idx).astype(jnp.float32)

    q = jax.lax.dot_general(
        onehot, w, (((1,), (0,)), ((), ())),
        preferred_element_type=jnp.float32)
    qst_ref[...] = x + (q - x)

    blk_sumsq = jnp.sum((q - x) ** 2)
    ones = jnp.ones((1, BLOCK_ROWS), dtype=jnp.float32)
    blk_counts = jax.lax.dot_general(
        ones, onehot, (((1,), (0,)), ((), ())),
        preferred_element_type=jnp.float32)

    @pl.when(i == 0)
    def _init():
        sumsq_ref[0, 0] = blk_sumsq
        counts_ref[...] = blk_counts

    @pl.when(i != 0)
    def _acc():
        sumsq_ref[0, 0] += blk_sumsq
        counts_ref[...] += blk_counts

    @pl.when(i == N_BLOCKS - 1)
    def _finalize():
        mse = sumsq_ref[0, 0] / jnp.float32(N_ROWS * DIM)
        loss_ref[...] = jnp.full((1, 1), mse + COMMITMENT_COST * mse,
                                 dtype=jnp.float32)
        probs = counts_ref[...] / jnp.float32(N_ROWS)
        ent = -jnp.sum(probs * jnp.log(probs + 1e-10))
        perp_ref[...] = jnp.full((1, 1), jnp.exp(ent), dtype=jnp.float32)


def kernel(inputs, weight):
    x = jnp.transpose(inputs, (0, 2, 3, 1))
    input_shape = x.shape
    flat = x.reshape(-1, DIM)
    distances = (jnp.sum(flat ** 2, axis=1, keepdims=True)
                 + jnp.sum(weight ** 2, axis=1)
                 - 2.0 * jnp.matmul(flat, weight.T))
    encoding_indices = jnp.argmin(distances, axis=1)
    encodings = jax.nn.one_hot(encoding_indices, N_EMBED, dtype=jnp.float32)

    qst, loss, perp = pl.pallas_call(
        _vq_tail_kernel,
        grid=(N_BLOCKS,),
        in_specs=[
            pl.BlockSpec((BLOCK_ROWS, 1), lambda i: (i, 0)),
            pl.BlockSpec((BLOCK_ROWS, DIM), lambda i: (i, 0)),
            pl.BlockSpec((N_EMBED, DIM), lambda i: (0, 0)),
        ],
        out_specs=[
            pl.BlockSpec((BLOCK_ROWS, DIM), lambda i: (i, 0)),
            pl.BlockSpec((1, 1), lambda i: (0, 0)),
            pl.BlockSpec((1, 1), lambda i: (0, 0)),
        ],
        out_shape=[
            jax.ShapeDtypeStruct((N_ROWS, DIM), jnp.float32),
            jax.ShapeDtypeStruct((1, 1), jnp.float32),
            jax.ShapeDtypeStruct((1, 1), jnp.float32),
        ],
        scratch_shapes=[
            pltpu.VMEM((1, N_EMBED), jnp.float32),
            pltpu.SMEM((1, 1), jnp.float32),
        ],
    )(encoding_indices.astype(jnp.int32).reshape(N_ROWS, 1), flat, weight)

    quantized_st = qst.reshape(input_shape)
    encoding_shape = input_shape[:-1] + (N_EMBED,)
    return (loss[0, 0],
            jnp.transpose(quantized_st, (0, 3, 1, 2)),
            perp[0, 0],
            encodings.reshape(encoding_shape))


# speedup vs baseline: 1.0396x; 1.0396x over previous
import jax
import jax.numpy as jnp
from jax.experimental import pallas as pl
from jax.experimental.pallas import tpu as pltpu

DIM = 32
N_EMBED = 8192
N_ROWS = 8192
COMMITMENT_COST = 0.25
BLOCK_ROWS = 1024
N_BLOCKS = N_ROWS // BLOCK_ROWS


def _vq_tail_kernel(idx_ref, x_ref, w_ref,
                    qst_ref, loss_ref, perp_ref,
                    counts_ref, sumsq_ref):
    i = pl.program_id(0)

    x = x_ref[...]
    w = w_ref[...]
    idx = idx_ref[...]

    iota = jax.lax.broadcasted_iota(jnp.int32, (BLOCK_ROWS, N_EMBED), 1)
    onehot = (iota == idx).astype(jnp.float32)

    q = jax.lax.dot_general(
        onehot, w, (((1,), (0,)), ((), ())),
        preferred_element_type=jnp.float32)
    qst_ref[...] = x + (q - x)

    blk_sumsq = jnp.sum((q - x) ** 2)
    blk_counts = jnp.sum(onehot, axis=0, keepdims=True)

    @pl.when(i == 0)
    def _init():
        sumsq_ref[0, 0] = blk_sumsq
        counts_ref[...] = blk_counts

    @pl.when(i != 0)
    def _acc():
        sumsq_ref[0, 0] += blk_sumsq
        counts_ref[...] += blk_counts

    @pl.when(i == N_BLOCKS - 1)
    def _finalize():
        mse = sumsq_ref[0, 0] / jnp.float32(N_ROWS * DIM)
        loss_ref[...] = jnp.full((1, 1), mse + COMMITMENT_COST * mse,
                                 dtype=jnp.float32)
        probs = counts_ref[...] / jnp.float32(N_ROWS)
        ent = -jnp.sum(probs * jnp.log(probs + 1e-10))
        perp_ref[...] = jnp.full((1, 1), jnp.exp(ent), dtype=jnp.float32)


def kernel(inputs, weight):
    x = jnp.transpose(inputs, (0, 2, 3, 1))
    input_shape = x.shape
    flat = x.reshape(-1, DIM)
    distances = (jnp.sum(flat ** 2, axis=1, keepdims=True)
                 + jnp.sum(weight ** 2, axis=1)
                 - 2.0 * jnp.matmul(flat, weight.T))
    encoding_indices = jnp.argmin(distances, axis=1)
    encodings = jax.nn.one_hot(encoding_indices, N_EMBED, dtype=jnp.float32)

    qst, loss, perp = pl.pallas_call(
        _vq_tail_kernel,
        grid=(N_BLOCKS,),
        in_specs=[
            pl.BlockSpec((BLOCK_ROWS, 1), lambda i: (i, 0)),
            pl.BlockSpec((BLOCK_ROWS, DIM), lambda i: (i, 0)),
            pl.BlockSpec((N_EMBED, DIM), lambda i: (0, 0)),
        ],
        out_specs=[
            pl.BlockSpec((BLOCK_ROWS, DIM), lambda i: (i, 0)),
            pl.BlockSpec((1, 1), lambda i: (0, 0)),
            pl.BlockSpec((1, 1), lambda i: (0, 0)),
        ],
        out_shape=[
            jax.ShapeDtypeStruct((N_ROWS, DIM), jnp.float32),
            jax.ShapeDtypeStruct((1, 1), jnp.float32),
            jax.ShapeDtypeStruct((1, 1), jnp.float32),
        ],
        scratch_shapes=[
            pltpu.VMEM((1, N_EMBED), jnp.float32),
            pltpu.SMEM((1, 1), jnp.float32),
        ],
    )(encoding_indices.astype(jnp.int32).reshape(N_ROWS, 1), flat, weight)

    quantized_st = qst.reshape(input_shape)
    encoding_shape = input_shape[:-1] + (N_EMBED,)
    return (loss[0, 0],
            jnp.transpose(quantized_st, (0, 3, 1, 2)),
            perp[0, 0],
            encodings.reshape(encoding_shape))
